# baseline (speedup 1.0000x reference)
import functools

import jax
import jax.numpy as jnp
from jax import lax
from jax.experimental import pallas as pl
from jax.experimental.pallas import tpu as pltpu
from jax.experimental.pallas import tpu_sc as plsc

D = 64
L = 16
NC, NS = 2, 16
NW = NC * NS
CHUNK = 128
GN = 5
AHEAD = 3


def _pe_row():
    i = jnp.arange(D // 2, dtype=jnp.float32)
    ij = i / jnp.power(10000.0, 2.0 * (i / D))
    sin_cos = jnp.stack([jnp.sin(ij), jnp.cos(ij)], axis=1)
    return jnp.reshape(sin_cos, (D,))


def _g_body(n_chunks_w, idx_hbm, pe_hbm, table_hbm, out_hbm,
            idx_v, pe_v, gbufs, tbufs, gsem, wsem):
    wid = lax.axis_index("s") * NC + lax.axis_index("c")
    n_tok_w = n_chunks_w * CHUNK
    base = wid * n_chunks_w
    pltpu.sync_copy(idx_hbm.at[pl.ds(wid * n_tok_w, n_tok_w)], idx_v)
    pltpu.sync_copy(pe_hbm, pe_v)
    pe_regs = [pe_v[pl.ds(L * t, L)] for t in range(D // L)]
    d_base = [lax.iota(jnp.int32, L) + t * L for t in range(D // L)]

    def fire_gather(chunk, b):
        pltpu.async_copy(table_hbm.at[idx_v.at[pl.ds(chunk * CHUNK, CHUNK)]],
                         gbufs.at[b], gsem.at[b])

    def wait_gather(b):
        pltpu.make_async_copy(table_hbm.at[idx_v.at[pl.ds(0, CHUNK)]],
                              gbufs.at[b], gsem.at[b]).wait()

    def fire_write(chunk, b):
        gc = base + chunk
        s = gc // (1024 // CHUNK)
        b0 = (gc % (1024 // CHUNK)) * CHUNK
        pltpu.async_copy(tbufs.at[b, :, pl.ds(0, CHUNK)],
                         out_hbm.at[s, :, pl.ds(b0, CHUNK)], wsem.at[b])

    def wait_write(b):
        pltpu.make_async_copy(tbufs.at[b, :, pl.ds(0, CHUNK)],
                              out_hbm.at[0, :, pl.ds(0, CHUNK)],
                              wsem.at[b]).wait()

    for j in range(AHEAD):
        fire_gather(j, j % GN)

    @pl.loop(0, n_chunks_w, step=GN)
    def _group(j0):
        for b in range(GN):
            j = j0 + b
            k = j + AHEAD
            kb = (b + AHEAD) % GN

            @pl.when(k < n_chunks_w)
            def _():
                fire_gather(k, kb)

            wait_gather(b)

            @pl.when(j >= GN)
            def _():
                wait_write(b)

            @plsc.parallel_loop(0, CHUNK, unroll=4)
            def _tok(c):
                c_vec = jnp.full((L,), c, dtype=jnp.int32)
                for t in range(D // L):
                    v = gbufs[b, c, pl.ds(t * L, L)] + pe_regs[t]
                    plsc.store_scatter(tbufs.at[b], [d_base[t], c_vec], v)

            fire_write(j, b)

    for b in range(GN):
        wait_write(b)


def kernel(inputs, table):
    bsz, seq = inputs.shape
    n = bsz * seq
    assert bsz % CHUNK == 0 and n % (NW * CHUNK) == 0
    n_chunks_w = n // (NW * CHUNK)
    assert n_chunks_w % GN == 0
    idx = inputs.T.reshape(-1).astype(jnp.int32)
    pe = _pe_row()
    mesh = plsc.VectorSubcoreMesh(core_axis_name="c", subcore_axis_name="s")

    gather_k = pl.kernel(
        functools.partial(_g_body, n_chunks_w),
        out_type=jax.ShapeDtypeStruct((seq, D, bsz), jnp.float32),
        mesh=mesh,
        compiler_params=pltpu.CompilerParams(use_tc_tiling_on_sc=False,
                                             needs_layout_passes=False),
        scratch_types=[
            pltpu.VMEM((n_chunks_w * CHUNK,), jnp.int32),
            pltpu.VMEM((D,), jnp.float32),
            pltpu.VMEM((GN, CHUNK, D), jnp.float32),
            pltpu.VMEM((GN, D, CHUNK + 1), jnp.float32),
            pltpu.SemaphoreType.DMA((GN,)),
            pltpu.SemaphoreType.DMA((GN,)),
        ],
    )
    out = gather_k(idx, pe, table)
    return jnp.transpose(out, (2, 0, 1))
